# Initial kernel scaffold; baseline (speedup 1.0000x reference)
import jax
import jax.numpy as jnp
from jax.experimental import pallas as pl


def kernel(unary, binary, deltas, index1, index2):
    raise NotImplementedError("write your pallas kernel here")


# speedup vs baseline: 6.9617x; 6.9617x over previous
import functools

import jax
import jax.numpy as jnp
from jax import lax
from jax.experimental import pallas as pl
from jax.experimental.pallas import tpu as pltpu
from jax.experimental.pallas import tpu_sc as plsc

_NODES = 50000
_EDGES = 1600000
_F = 16
_NC = 2
_NS = 16
_NODES_PAD = 50048
_SLICE = _NODES_PAD // _NS
_CHUNK = 125
_K = 8
_STEP = _K * _CHUNK
_GRID = _EDGES // _STEP


def _sc_scatter(deltas, idx1, idx2, zeros):
    mesh = plsc.VectorSubcoreMesh(core_axis_name="core", subcore_axis_name="subcore")

    @functools.partial(
        pl.kernel,
        out_type=jax.ShapeDtypeStruct((_NC, _NODES_PAD, _F), jnp.float32),
        mesh=mesh,
        scratch_types=[pltpu.VMEM_SHARED((_NODES_PAD, _F), jnp.float32)],
        compiler_params=pltpu.CompilerParams(use_tc_tiling_on_sc=False),
    )
    def k(deltas_hbm, idx1_hbm, idx2_hbm, zeros_hbm, partials_hbm, acc):
        c = lax.axis_index("core")
        s = lax.axis_index("subcore")
        r0 = s * _SLICE
        pltpu.sync_copy(zeros_hbm.at[pl.ds(r0, _SLICE)], acc.at[pl.ds(r0, _SLICE)])
        plsc.subcore_barrier()

        def body(ux_v, i1_v, uy_v, i2_v):
            for j in range(_K):
                rows = pl.ds(j * _CHUNK, _CHUNK)
                pltpu.sync_copy(ux_v.at[rows], acc.at[i1_v.at[j]], add=True)
                pltpu.sync_copy(uy_v.at[rows], acc.at[i2_v.at[j]], add=True)

        pltpu.emit_pipeline(
            body,
            grid=(_GRID,),
            in_specs=[
                pl.BlockSpec((_STEP, _F), lambda i: (i, 0)),
                pl.BlockSpec((_K, _CHUNK), lambda i: (i, 0)),
                pl.BlockSpec((_STEP, _F), lambda i: (i, 1)),
                pl.BlockSpec((_K, _CHUNK), lambda i: (i, 0)),
            ],
            out_specs=[],
            core_axis_name=("core", "subcore"),
            dimension_semantics=(pltpu.PARALLEL,),
        )(deltas_hbm, idx1_hbm, deltas_hbm, idx2_hbm)

        plsc.subcore_barrier()
        pltpu.sync_copy(acc.at[pl.ds(r0, _SLICE)],
                        partials_hbm.at[c, pl.ds(r0, _SLICE)])

    return k(deltas, idx1, idx2, zeros)


def _tc_sum(partials):
    def body(p_ref, o_ref):
        o_ref[...] = p_ref[0] + p_ref[1]

    r = partials.shape[1]
    return pl.pallas_call(
        body,
        out_shape=jax.ShapeDtypeStruct((r, 128), jnp.float32),
    )(partials)


def kernel(unary, binary, deltas, index1, index2):
    idx1 = index1.reshape(_GRID * _K, _CHUNK)
    idx2 = index2.reshape(_GRID * _K, _CHUNK)
    zeros = jnp.zeros((_NODES_PAD, _F), jnp.float32)
    partials = _sc_scatter(deltas, idx1, idx2, zeros)
    out1 = _tc_sum(partials.reshape(_NC, _NODES_PAD * _F // 128, 128))
    out1 = out1.reshape(_NODES_PAD, _F)[:_NODES]
    b = deltas[:, 2 * _F:]
    return (out1, b)
